# Initial kernel scaffold; baseline (speedup 1.0000x reference)
import jax
import jax.numpy as jnp
from jax import lax
from jax.experimental import pallas as pl
from jax.experimental.pallas import tpu as pltpu
from jax.experimental.pallas import tpu_sc as plsc

_L = 16
_SEQ = 2048
_TILES_PER_ROW = 8
_CHUNK = _SEQ // _TILES_PER_ROW
_GROUPS = _CHUNK // _L
_ROW_LEN = 8192


def _sc_body(x_hbm, y_hbm, len_hbm, out_hbm,
             row_v, idx_v, len_v, acc_v, all_v, tmp_v, out_v, shared):
  c = lax.axis_index("c")
  s = lax.axis_index("s")

  @pl.when(c == 0)
  def _core0():
    row = s // _TILES_PER_ROW
    base = pl.multiple_of((s % _TILES_PER_ROW) * _CHUNK, _CHUNK)

    pltpu.sync_copy(x_hbm.at[row], row_v)
    pltpu.sync_copy(y_hbm.at[row, pl.ds(base, _CHUNK)], idx_v)
    pltpu.sync_copy(len_hbm, len_v)

    iota = lax.iota(jnp.int32, _L)
    row_vec = jnp.zeros((_L,), jnp.int32) + row
    lrow = plsc.load_gather(len_v, [row_vec])

    acc = jnp.full((_L,), 1.0, jnp.float32)
    for g in range(_GROUPS):
      idx = idx_v[pl.ds(g * _L, _L)]
      vals = plsc.load_gather(row_v, [idx])
      pos = base + (g * _L) + iota
      acc = acc * jnp.where(pos < lrow, 1.0 - vals, 1.0)

    acc_v[...] = acc
    pltpu.sync_copy(acc_v, shared.at[s])
    plsc.subcore_barrier()

    @pl.when(s == 0)
    def _finalize():
      pltpu.sync_copy(shared, all_v)
      p0 = all_v[0, :]
      p1 = all_v[_TILES_PER_ROW, :]
      for t in range(1, _TILES_PER_ROW):
        p0 = p0 * all_v[t, :]
        p1 = p1 * all_v[_TILES_PER_ROW + t, :]
      for k in (1, 2, 4, 8):
        perm = iota ^ k
        tmp_v[...] = p0
        p0 = p0 * plsc.load_gather(tmp_v, [perm])
        tmp_v[...] = p1
        p1 = p1 * plsc.load_gather(tmp_v, [perm])
      out_v[...] = 2.0 - p0 - p1
      pltpu.sync_copy(out_v, out_hbm)


@jax.jit
def kernel(x, y, lengths):
  mesh = plsc.VectorSubcoreMesh(core_axis_name="c", subcore_axis_name="s")
  out = pl.kernel(
      _sc_body,
      out_type=jax.ShapeDtypeStruct((_L,), jnp.float32),
      mesh=mesh,
      scratch_types=[
          pltpu.VMEM((_ROW_LEN,), jnp.float32),
          pltpu.VMEM((_CHUNK,), jnp.int32),
          pltpu.VMEM((_L,), jnp.int32),
          pltpu.VMEM((_L,), jnp.float32),
          pltpu.VMEM((_L, _L), jnp.float32),
          pltpu.VMEM((_L,), jnp.float32),
          pltpu.VMEM((_L,), jnp.float32),
          pltpu.VMEM_SHARED((_L, _L), jnp.float32),
      ],
  )(x, y, lengths)
  return out[:1]


# speedup vs baseline: 1.3746x; 1.3746x over previous
import jax
import jax.numpy as jnp
from jax import lax
from jax.experimental import pallas as pl
from jax.experimental.pallas import tpu as pltpu
from jax.experimental.pallas import tpu_sc as plsc

_L = 16
_SEQ = 2048
_TILES_PER_ROW = 8
_CHUNK = _SEQ // _TILES_PER_ROW
_GROUPS = _CHUNK // _L
_ROW_LEN = 8192


def _sc_body(x_hbm, y_hbm, len_hbm, out_hbm,
             row_v, idx_v, len_v, acc_v, all_v, tmp_v, out_v, shared):
  c = lax.axis_index("c")
  s = lax.axis_index("s")

  @pl.when(c == 0)
  def _core0():
    row = s // _TILES_PER_ROW
    base = pl.multiple_of((s % _TILES_PER_ROW) * _CHUNK, _CHUNK)

    pltpu.sync_copy(x_hbm.at[row], row_v)
    pltpu.sync_copy(y_hbm.at[row, pl.ds(base, _CHUNK)], idx_v)
    pltpu.sync_copy(len_hbm, len_v)

    iota = lax.iota(jnp.int32, _L)
    row_vec = jnp.zeros((_L,), jnp.int32) + row
    lrow = plsc.load_gather(len_v, [row_vec])

    acc = jnp.full((_L,), 1.0, jnp.float32)
    for g in range(_GROUPS):
      idx = idx_v[pl.ds(g * _L, _L)]
      vals = plsc.load_gather(row_v, [idx])
      pos = base + (g * _L) + iota
      acc = acc * jnp.where(pos < lrow, 1.0 - vals, 1.0)

    acc_v[...] = acc
    pltpu.sync_copy(acc_v, shared.at[s])
    plsc.subcore_barrier()

    @pl.when(s == 0)
    def _finalize():
      pltpu.sync_copy(shared, all_v)
      p0 = all_v[0, :]
      p1 = all_v[_TILES_PER_ROW, :]
      for t in range(1, _TILES_PER_ROW):
        p0 = p0 * all_v[t, :]
        p1 = p1 * all_v[_TILES_PER_ROW + t, :]
      for k in (1, 2, 4, 8):
        perm = iota ^ k
        tmp_v[...] = p0
        p0 = p0 * plsc.load_gather(tmp_v, [perm])
        tmp_v[...] = p1
        p1 = p1 * plsc.load_gather(tmp_v, [perm])
      out_v[...] = 2.0 - p0 - p1
      pltpu.sync_copy(out_v, out_hbm)


@jax.jit
def kernel(x, y, lengths):
  mesh = plsc.VectorSubcoreMesh(core_axis_name="c", subcore_axis_name="s")
  out = pl.kernel(
      _sc_body,
      out_type=jax.ShapeDtypeStruct((_L,), jnp.float32),
      mesh=mesh,
      compiler_params=pltpu.CompilerParams(needs_layout_passes=False),
      scratch_types=[
          pltpu.VMEM((_ROW_LEN,), jnp.float32),
          pltpu.VMEM((_CHUNK,), jnp.int32),
          pltpu.VMEM((_L,), jnp.int32),
          pltpu.VMEM((_L,), jnp.float32),
          pltpu.VMEM((_L, _L), jnp.float32),
          pltpu.VMEM((_L,), jnp.float32),
          pltpu.VMEM((_L,), jnp.float32),
          pltpu.VMEM_SHARED((_L, _L), jnp.float32),
      ],
  )(x, y, lengths)
  return out[:1]


# speedup vs baseline: 1.5607x; 1.1354x over previous
import jax
import jax.numpy as jnp
from jax import lax
from jax.experimental import pallas as pl
from jax.experimental.pallas import tpu as pltpu
from jax.experimental.pallas import tpu_sc as plsc

_L = 16
_SEQ = 2048
_TILES_PER_ROW = 8
_CHUNK = _SEQ // _TILES_PER_ROW
_GROUPS = _CHUNK // _L
_ROW_LEN = 8192


def _sc_body(x_hbm, y_hbm, len_hbm, out_hbm,
             row_v, idx_v, len_v, acc_v, all_v, tmp_v, out_v, shared,
             sem_x, sem_y, sem_l):
  s = lax.axis_index("s")

  if True:
    row = s // _TILES_PER_ROW
    base = pl.multiple_of((s % _TILES_PER_ROW) * _CHUNK, _CHUNK)

    cp_x = pltpu.async_copy(x_hbm.at[row], row_v, sem_x)
    cp_y = pltpu.async_copy(y_hbm.at[row, pl.ds(base, _CHUNK)], idx_v, sem_y)
    cp_l = pltpu.async_copy(len_hbm, len_v, sem_l)
    cp_l.wait()
    cp_y.wait()
    cp_x.wait()

    iota = lax.iota(jnp.int32, _L)
    row_vec = jnp.zeros((_L,), jnp.int32) + row
    lrow = plsc.load_gather(len_v, [row_vec])

    acc = jnp.full((_L,), 1.0, jnp.float32)
    for g in range(_GROUPS):
      idx = idx_v[pl.ds(g * _L, _L)]
      vals = plsc.load_gather(row_v, [idx])
      pos = base + (g * _L) + iota
      acc = acc * jnp.where(pos < lrow, 1.0 - vals, 1.0)

    acc_v[...] = acc
    pltpu.sync_copy(acc_v, shared.at[s])
    plsc.subcore_barrier()

    @pl.when(s == 0)
    def _finalize():
      pltpu.sync_copy(shared, all_v)
      p0 = all_v[0, :]
      p1 = all_v[_TILES_PER_ROW, :]
      for t in range(1, _TILES_PER_ROW):
        p0 = p0 * all_v[t, :]
        p1 = p1 * all_v[_TILES_PER_ROW + t, :]
      for k in (1, 2, 4, 8):
        perm = iota ^ k
        tmp_v[...] = p0
        p0 = p0 * plsc.load_gather(tmp_v, [perm])
        tmp_v[...] = p1
        p1 = p1 * plsc.load_gather(tmp_v, [perm])
      out_v[...] = 2.0 - p0 - p1
      pltpu.sync_copy(out_v, out_hbm)


@jax.jit
def kernel(x, y, lengths):
  mesh = plsc.VectorSubcoreMesh(
      core_axis_name="c", subcore_axis_name="s", num_cores=1)
  out = pl.kernel(
      _sc_body,
      out_type=jax.ShapeDtypeStruct((_L,), jnp.float32),
      mesh=mesh,
      compiler_params=pltpu.CompilerParams(needs_layout_passes=False),
      scratch_types=[
          pltpu.VMEM((_ROW_LEN,), jnp.float32),
          pltpu.VMEM((_CHUNK,), jnp.int32),
          pltpu.VMEM((_L,), jnp.int32),
          pltpu.VMEM((_L,), jnp.float32),
          pltpu.VMEM((_L, _L), jnp.float32),
          pltpu.VMEM((_L,), jnp.float32),
          pltpu.VMEM((_L,), jnp.float32),
          pltpu.VMEM_SHARED((_L, _L), jnp.float32),
          pltpu.SemaphoreType.DMA,
          pltpu.SemaphoreType.DMA,
          pltpu.SemaphoreType.DMA,
      ],
  )(x, y, lengths)
  return out[:1]
